# Initial kernel scaffold; baseline (speedup 1.0000x reference)
import jax
import jax.numpy as jnp
from jax.experimental import pallas as pl


def kernel(x, lut_weight):
    raise NotImplementedError("write your pallas kernel here")


# speedup vs baseline: 1.8304x; 1.8304x over previous
import functools

import jax
import jax.numpy as jnp
from jax import lax
from jax.experimental import pallas as pl
from jax.experimental.pallas import tpu as pltpu
from jax.experimental.pallas import tpu_sc as plsc

_D = 64
_NC = 2
_NS = 16
_NW = _NC * _NS


@functools.lru_cache(maxsize=None)
def _gather_kernel(B, C):
    b_per_w = B // _NW
    nchunks = b_per_w // C
    mesh = plsc.VectorSubcoreMesh(core_axis_name="c", subcore_axis_name="s")

    @functools.partial(
        pl.kernel,
        out_type=jax.ShapeDtypeStruct((B, _D), jnp.float32),
        mesh=mesh,
        scratch_types=[
            pltpu.VMEM((C,), jnp.int32),
            pltpu.VMEM((C, _D), jnp.float32),
            pltpu.SemaphoreType.DMA,
        ],
        compiler_params=pltpu.CompilerParams(use_tc_tiling_on_sc=False),
    )
    def k(table_hbm, idx_hbm, out_hbm, idx_v, rows_v, sem):
        wid = lax.axis_index("s") * _NC + lax.axis_index("c")
        base = wid * b_per_w

        @pl.loop(0, nchunks)
        def _(g):
            off = base + g * C
            pltpu.sync_copy(idx_hbm.at[pl.ds(off, C)], idx_v)
            pltpu.async_copy(table_hbm.at[idx_v], rows_v, sem).wait()
            pltpu.sync_copy(rows_v, out_hbm.at[pl.ds(off, C)])

    return k


def kernel(x, lut_weight):
    nb, nh = x.shape
    B = nb * nh
    flat = x.reshape(B)
    out = _gather_kernel(B, 800)(lut_weight, flat)
    return out.reshape(nb, nh, _D)


# speedup vs baseline: 1.8752x; 1.0244x over previous
import functools

import jax
import jax.numpy as jnp
from jax import lax
from jax.experimental import pallas as pl
from jax.experimental.pallas import tpu as pltpu
from jax.experimental.pallas import tpu_sc as plsc

_D = 64
_NC = 2
_NS = 16
_NW = _NC * _NS


@functools.lru_cache(maxsize=None)
def _gather_kernel(B, C):
    b_per_w = B // _NW
    nchunks = b_per_w // C
    assert nchunks % 2 == 0
    mesh = plsc.VectorSubcoreMesh(core_axis_name="c", subcore_axis_name="s")

    @functools.partial(
        pl.kernel,
        out_type=jax.ShapeDtypeStruct((B, _D), jnp.float32),
        mesh=mesh,
        scratch_types=[
            pltpu.VMEM((b_per_w,), jnp.int32),
            pltpu.VMEM((C, _D), jnp.float32),
            pltpu.VMEM((C, _D), jnp.float32),
            pltpu.SemaphoreType.DMA,
            pltpu.SemaphoreType.DMA,
            pltpu.SemaphoreType.DMA,
            pltpu.SemaphoreType.DMA,
        ],
        compiler_params=pltpu.CompilerParams(use_tc_tiling_on_sc=False),
    )
    def k(table_hbm, idx_hbm, out_hbm, idx_v, rows0, rows1, g0, g1, s0, s1):
        wid = lax.axis_index("s") * _NC + lax.axis_index("c")
        base = wid * b_per_w
        rows = (rows0, rows1)
        gsem = (g0, g1)
        ssem = (s0, s1)

        pltpu.sync_copy(idx_hbm.at[pl.ds(base, b_per_w)], idx_v)
        pltpu.async_copy(table_hbm.at[idx_v.at[pl.ds(0, C)]], rows[0], gsem[0])
        pltpu.async_copy(table_hbm.at[idx_v.at[pl.ds(C, C)]], rows[1], gsem[1])

        @pl.loop(0, nchunks // 2)
        def _(i):
            for b in range(2):
                g = 2 * i + b
                pltpu.make_async_copy(
                    table_hbm.at[pl.ds(0, C)], rows[b], gsem[b]
                ).wait()
                st = pltpu.async_copy(
                    rows[b], out_hbm.at[pl.ds(base + g * C, C)], ssem[b]
                )
                st.wait()

                @pl.when(g + 2 < nchunks)
                def _():
                    pltpu.async_copy(
                        table_hbm.at[idx_v.at[pl.ds((g + 2) * C, C)]],
                        rows[b],
                        gsem[b],
                    )

    return k


def kernel(x, lut_weight):
    nb, nh = x.shape
    B = nb * nh
    flat = x.reshape(B)
    out = _gather_kernel(B, 800)(lut_weight, flat)
    return out.reshape(nb, nh, _D)
